# Initial kernel scaffold; baseline (speedup 1.0000x reference)
import jax
import jax.numpy as jnp
from jax.experimental import pallas as pl


def kernel(x, edge_attr, W, b, edge_index):
    raise NotImplementedError("write your pallas kernel here")


# speedup vs baseline: 4.4959x; 4.4959x over previous
import functools

import jax
import jax.numpy as jnp
from jax import lax
from jax.experimental import pallas as pl
from jax.experimental.pallas import tpu as pltpu
from jax.experimental.pallas import tpu_sc as plsc

N_NODES = 10000
N_EDGES = 160000
TWO_E = 2 * N_EDGES
D_FEAT = 256
HALF = 8
ROW_W = 16

NUM_CORES = 2
NUM_SUBCORES = 16
NUM_WORKERS = NUM_CORES * NUM_SUBCORES
BLK = 128
BPW = 80
TOT = NUM_WORKERS * BPW * BLK
N_BLOCKS = TOT // BLK

NP = 10112
RPT = NP // NUM_SUBCORES


def _worker_id():
    return lax.axis_index("c") * NUM_SUBCORES + lax.axis_index("s")


_MESH = plsc.VectorSubcoreMesh(core_axis_name="c", subcore_axis_name="s")
_SC_PARAMS = pltpu.CompilerParams(use_tc_tiling_on_sc=False)


@functools.partial(
    pl.kernel,
    out_type=jax.ShapeDtypeStruct((NUM_CORES, NP, ROW_W), jnp.float32),
    mesh=_MESH,
    scratch_types=[
        pltpu.VMEM((BPW, BLK), jnp.int32),
        pltpu.VMEM((BLK, ROW_W), jnp.float32),
        pltpu.VMEM((RPT, ROW_W), jnp.float32),
        pltpu.VMEM_SHARED((NP, ROW_W), jnp.float32),
    ],
    compiler_params=_SC_PARAMS,
)
def _scatter_add_vals(vals_hbm, idx_hbm, zeros_hbm, out_hbm,
                      idx_v, vbuf, tmp, acc_sh):
    cid = lax.axis_index("c")
    sid = lax.axis_index("s")
    w = _worker_id()
    z0 = sid * RPT
    pltpu.sync_copy(zeros_hbm.at[pl.ds(z0, RPT)], acc_sh.at[pl.ds(z0, RPT)])
    pltpu.sync_copy(idx_hbm.at[pl.ds(w * BPW, BPW)], idx_v)
    plsc.subcore_barrier()

    def body(j, carry):
        blk = w * BPW + j
        pltpu.sync_copy(vals_hbm.at[pl.ds(blk * BLK, BLK)], vbuf)
        pltpu.sync_copy(vbuf, acc_sh.at[idx_v.at[j]], add=True)
        return carry

    lax.fori_loop(0, BPW, body, 0)
    plsc.subcore_barrier()
    pltpu.sync_copy(acc_sh.at[pl.ds(z0, RPT)], tmp)
    pltpu.sync_copy(tmp, out_hbm.at[cid, pl.ds(z0, RPT)])


@functools.partial(
    pl.kernel,
    out_type=jax.ShapeDtypeStruct((NUM_CORES, NP, ROW_W), jnp.float32),
    mesh=_MESH,
    scratch_types=[
        pltpu.VMEM((BPW, BLK), jnp.int32),
        pltpu.VMEM((BPW, BLK), jnp.int32),
        pltpu.VMEM((BLK, ROW_W), jnp.float32),
        pltpu.VMEM((RPT, ROW_W), jnp.float32),
        pltpu.VMEM_SHARED((NP, ROW_W), jnp.float32),
    ],
    compiler_params=_SC_PARAMS,
)
def _gather_scatter_add(agg_hbm, idxo_hbm, idxi_hbm, zeros_hbm, out_hbm,
                        idxo_v, idxi_v, gbuf, tmp, sums_sh):
    cid = lax.axis_index("c")
    sid = lax.axis_index("s")
    w = _worker_id()
    z0 = sid * RPT
    pltpu.sync_copy(zeros_hbm.at[pl.ds(z0, RPT)], sums_sh.at[pl.ds(z0, RPT)])
    pltpu.sync_copy(idxo_hbm.at[pl.ds(w * BPW, BPW)], idxo_v)
    pltpu.sync_copy(idxi_hbm.at[pl.ds(w * BPW, BPW)], idxi_v)
    plsc.subcore_barrier()

    def body(j, carry):
        pltpu.sync_copy(agg_hbm.at[idxo_v.at[j]], gbuf)
        pltpu.sync_copy(gbuf, sums_sh.at[idxi_v.at[j]], add=True)
        return carry

    lax.fori_loop(0, BPW, body, 0)
    plsc.subcore_barrier()
    pltpu.sync_copy(sums_sh.at[pl.ds(z0, RPT)], tmp)
    pltpu.sync_copy(tmp, out_hbm.at[cid, pl.ds(z0, RPT)])


ROWS_BLK = 400


def _dense_body(x_ref, acc_ref, s0_ref, s1_ref, w_ref, b_ref, out_ref):
    acc = acc_ref[...]
    sums = s0_ref[...] + s1_ref[...]
    cnt = jnp.maximum(acc[:, HALF:HALF + 1], 1.0)
    navg = sums[:, :HALF] / cnt
    xb = x_ref[...]
    w = w_ref[...]
    out = jnp.dot(navg, w[:HALF, :], preferred_element_type=jnp.float32)
    out += jnp.dot(xb, w[HALF:, :], preferred_element_type=jnp.float32)
    out_ref[...] = out + b_ref[...]


def kernel(x, edge_attr, W, b, edge_index):
    senders = edge_index[0]
    receivers = edge_index[1]
    pad_e = TOT - TWO_E
    idx_in = jnp.concatenate(
        [senders, receivers,
         jnp.full((pad_e,), NP - 1, dtype=jnp.int32)]).reshape(N_BLOCKS, BLK)
    idx_out = jnp.concatenate(
        [receivers, senders,
         jnp.zeros((pad_e,), dtype=jnp.int32)]).reshape(N_BLOCKS, BLK)

    two_ea = jnp.concatenate([edge_attr[:, :HALF], edge_attr[:, HALF:]], axis=0)
    vals = jnp.concatenate(
        [two_ea,
         jnp.ones((TWO_E, 1), dtype=jnp.float32),
         jnp.zeros((TWO_E, ROW_W - HALF - 1), dtype=jnp.float32)], axis=1)
    vals = jnp.concatenate(
        [vals, jnp.zeros((pad_e, ROW_W), dtype=jnp.float32)], axis=0)

    zeros_np = jnp.zeros((NP, ROW_W), dtype=jnp.float32)

    acc_pair = _scatter_add_vals(vals, idx_in, zeros_np)
    agg = (acc_pair[0, :N_NODES] + acc_pair[1, :N_NODES])

    sums_pair = _gather_scatter_add(agg, idx_out, idx_in, zeros_np)

    grid = N_NODES // ROWS_BLK
    out = pl.pallas_call(
        _dense_body,
        grid=(grid,),
        in_specs=[
            pl.BlockSpec((ROWS_BLK, D_FEAT), lambda i: (i, 0)),
            pl.BlockSpec((ROWS_BLK, ROW_W), lambda i: (i, 0)),
            pl.BlockSpec((ROWS_BLK, ROW_W), lambda i: (i, 0)),
            pl.BlockSpec((ROWS_BLK, ROW_W), lambda i: (i, 0)),
            pl.BlockSpec((D_FEAT + HALF, D_FEAT), lambda i: (0, 0)),
            pl.BlockSpec((1, D_FEAT), lambda i: (0, 0)),
        ],
        out_specs=pl.BlockSpec((ROWS_BLK, D_FEAT), lambda i: (i, 0)),
        out_shape=jax.ShapeDtypeStruct((N_NODES, D_FEAT), jnp.float32),
    )(x, agg, sums_pair[0, :N_NODES], sums_pair[1, :N_NODES], W,
      b.reshape(1, D_FEAT))
    return out


# speedup vs baseline: 5.2333x; 1.1640x over previous
import functools

import jax
import jax.numpy as jnp
from jax import lax
from jax.experimental import pallas as pl
from jax.experimental.pallas import tpu as pltpu
from jax.experimental.pallas import tpu_sc as plsc

N_NODES = 10000
N_EDGES = 160000
TWO_E = 2 * N_EDGES
D_FEAT = 256
HALF = 8
ROW_W = 16

NUM_CORES = 2
NUM_SUBCORES = 16
NUM_WORKERS = NUM_CORES * NUM_SUBCORES
BLK = 128
BPW = 80
TOT = NUM_WORKERS * BPW * BLK
N_BLOCKS = TOT // BLK

NP = 10112
RPT = NP // NUM_SUBCORES


def _worker_id():
    return lax.axis_index("c") * NUM_SUBCORES + lax.axis_index("s")


_MESH = plsc.VectorSubcoreMesh(core_axis_name="c", subcore_axis_name="s")
_SC_PARAMS = pltpu.CompilerParams(use_tc_tiling_on_sc=False)


GRP = 8
NGRP = BPW // GRP


@functools.partial(
    pl.kernel,
    out_type=jax.ShapeDtypeStruct((NUM_CORES, NP, ROW_W), jnp.float32),
    mesh=_MESH,
    scratch_types=[
        pltpu.VMEM((BPW, BLK), jnp.int32),
        pltpu.VMEM((GRP * BLK, ROW_W), jnp.float32),
        pltpu.VMEM((RPT, ROW_W), jnp.float32),
        pltpu.VMEM_SHARED((NP, ROW_W), jnp.float32),
        pltpu.SemaphoreType.DMA,
    ],
    compiler_params=_SC_PARAMS,
)
def _scatter_add_vals(vals_hbm, idx_hbm, zeros_hbm, out_hbm,
                      idx_v, vbuf, tmp, acc_sh, sem_st):
    cid = lax.axis_index("c")
    sid = lax.axis_index("s")
    w = _worker_id()
    z0 = sid * RPT
    pltpu.sync_copy(zeros_hbm.at[pl.ds(z0, RPT)], acc_sh.at[pl.ds(z0, RPT)])
    pltpu.sync_copy(idx_hbm.at[pl.ds(w * BPW, BPW)], idx_v)
    plsc.subcore_barrier()

    def group(g, carry):
        blk0 = (w * BPW + g * GRP) * BLK
        pltpu.sync_copy(vals_hbm.at[pl.ds(blk0, GRP * BLK)], vbuf)
        descs = [
            pltpu.async_copy(vbuf.at[pl.ds(bq * BLK, BLK)],
                             acc_sh.at[idx_v.at[g * GRP + bq]],
                             sem_st, add=True)
            for bq in range(GRP)
        ]
        for d in descs:
            d.wait()
        return carry

    lax.fori_loop(0, NGRP, group, 0)
    plsc.subcore_barrier()
    pltpu.sync_copy(acc_sh.at[pl.ds(z0, RPT)], tmp)
    pltpu.sync_copy(tmp, out_hbm.at[cid, pl.ds(z0, RPT)])


@functools.partial(
    pl.kernel,
    out_type=jax.ShapeDtypeStruct((NUM_CORES, NP, ROW_W), jnp.float32),
    mesh=_MESH,
    scratch_types=[
        pltpu.VMEM((BPW, BLK), jnp.int32),
        pltpu.VMEM((BPW, BLK), jnp.int32),
        pltpu.VMEM((GRP * BLK, ROW_W), jnp.float32),
        pltpu.VMEM((RPT, ROW_W), jnp.float32),
        pltpu.VMEM_SHARED((NP, ROW_W), jnp.float32),
        pltpu.SemaphoreType.DMA,
        pltpu.SemaphoreType.DMA,
    ],
    compiler_params=_SC_PARAMS,
)
def _gather_scatter_add(agg_hbm, idxo_hbm, idxi_hbm, zeros_hbm, out_hbm,
                        idxo_v, idxi_v, gbuf, tmp, sums_sh, sem_g, sem_s):
    cid = lax.axis_index("c")
    sid = lax.axis_index("s")
    w = _worker_id()
    z0 = sid * RPT
    pltpu.sync_copy(zeros_hbm.at[pl.ds(z0, RPT)], sums_sh.at[pl.ds(z0, RPT)])
    pltpu.sync_copy(idxo_hbm.at[pl.ds(w * BPW, BPW)], idxo_v)
    pltpu.sync_copy(idxi_hbm.at[pl.ds(w * BPW, BPW)], idxi_v)
    plsc.subcore_barrier()

    def group(g, carry):
        descs = [
            pltpu.async_copy(agg_hbm.at[idxo_v.at[g * GRP + bq]],
                             gbuf.at[pl.ds(bq * BLK, BLK)], sem_g)
            for bq in range(GRP)
        ]
        for d in descs:
            d.wait()
        descs = [
            pltpu.async_copy(gbuf.at[pl.ds(bq * BLK, BLK)],
                             sums_sh.at[idxi_v.at[g * GRP + bq]],
                             sem_s, add=True)
            for bq in range(GRP)
        ]
        for d in descs:
            d.wait()
        return carry

    lax.fori_loop(0, NGRP, group, 0)
    plsc.subcore_barrier()
    pltpu.sync_copy(sums_sh.at[pl.ds(z0, RPT)], tmp)
    pltpu.sync_copy(tmp, out_hbm.at[cid, pl.ds(z0, RPT)])


ROWS_BLK = 400


def _dense_body(x_ref, acc_ref, s0_ref, s1_ref, w_ref, b_ref, out_ref):
    acc = acc_ref[...]
    sums = s0_ref[...] + s1_ref[...]
    cnt = jnp.maximum(acc[:, HALF:HALF + 1], 1.0)
    navg = sums[:, :HALF] / cnt
    xb = x_ref[...]
    w = w_ref[...]
    out = jnp.dot(navg, w[:HALF, :], preferred_element_type=jnp.float32)
    out += jnp.dot(xb, w[HALF:, :], preferred_element_type=jnp.float32)
    out_ref[...] = out + b_ref[...]


def kernel(x, edge_attr, W, b, edge_index):
    senders = edge_index[0]
    receivers = edge_index[1]
    pad_e = TOT - TWO_E
    idx_in = jnp.concatenate(
        [senders, receivers,
         jnp.full((pad_e,), NP - 1, dtype=jnp.int32)]).reshape(N_BLOCKS, BLK)
    idx_out = jnp.concatenate(
        [receivers, senders,
         jnp.zeros((pad_e,), dtype=jnp.int32)]).reshape(N_BLOCKS, BLK)

    two_ea = jnp.concatenate([edge_attr[:, :HALF], edge_attr[:, HALF:]], axis=0)
    vals = jnp.concatenate(
        [two_ea,
         jnp.ones((TWO_E, 1), dtype=jnp.float32),
         jnp.zeros((TWO_E, ROW_W - HALF - 1), dtype=jnp.float32)], axis=1)
    vals = jnp.concatenate(
        [vals, jnp.zeros((pad_e, ROW_W), dtype=jnp.float32)], axis=0)

    zeros_np = jnp.zeros((NP, ROW_W), dtype=jnp.float32)

    acc_pair = _scatter_add_vals(vals, idx_in, zeros_np)
    agg = (acc_pair[0, :N_NODES] + acc_pair[1, :N_NODES])

    sums_pair = _gather_scatter_add(agg, idx_out, idx_in, zeros_np)

    grid = N_NODES // ROWS_BLK
    out = pl.pallas_call(
        _dense_body,
        grid=(grid,),
        in_specs=[
            pl.BlockSpec((ROWS_BLK, D_FEAT), lambda i: (i, 0)),
            pl.BlockSpec((ROWS_BLK, ROW_W), lambda i: (i, 0)),
            pl.BlockSpec((ROWS_BLK, ROW_W), lambda i: (i, 0)),
            pl.BlockSpec((ROWS_BLK, ROW_W), lambda i: (i, 0)),
            pl.BlockSpec((D_FEAT + HALF, D_FEAT), lambda i: (0, 0)),
            pl.BlockSpec((1, D_FEAT), lambda i: (0, 0)),
        ],
        out_specs=pl.BlockSpec((ROWS_BLK, D_FEAT), lambda i: (i, 0)),
        out_shape=jax.ShapeDtypeStruct((N_NODES, D_FEAT), jnp.float32),
    )(x, agg, sums_pair[0, :N_NODES], sums_pair[1, :N_NODES], W,
      b.reshape(1, D_FEAT))
    return out


# speedup vs baseline: 11.1434x; 2.1293x over previous
import functools

import jax
import jax.numpy as jnp
from jax import lax
from jax.experimental import pallas as pl
from jax.experimental.pallas import tpu as pltpu
from jax.experimental.pallas import tpu_sc as plsc

N_NODES = 10000
N_EDGES = 160000
TWO_E = 2 * N_EDGES
D_FEAT = 256
HALF = 8
ROW_W = 16

NUM_CORES = 2
NUM_SUBCORES = 16
NUM_WORKERS = NUM_CORES * NUM_SUBCORES
BLK = 125
GRP = 8

BPW_A = N_EDGES // (NUM_WORKERS * BLK)
NBLK_A = N_EDGES // BLK
NGRP_A = BPW_A // GRP

BPW_B = TWO_E // (NUM_WORKERS * BLK)
NBLK_B = TWO_E // BLK
NGRP_B = BPW_B // GRP

NP = 10112
RPT = NP // NUM_SUBCORES


def _worker_id():
    return lax.axis_index("c") * NUM_SUBCORES + lax.axis_index("s")


_MESH = plsc.VectorSubcoreMesh(core_axis_name="c", subcore_axis_name="s")
_SC_PARAMS = pltpu.CompilerParams(use_tc_tiling_on_sc=False)


@functools.partial(
    pl.kernel,
    out_type=jax.ShapeDtypeStruct((NUM_CORES, 2, NP, ROW_W), jnp.float32),
    mesh=_MESH,
    scratch_types=[
        pltpu.VMEM((BPW_A, BLK), jnp.int32),
        pltpu.VMEM((BPW_A, BLK), jnp.int32),
        pltpu.VMEM((GRP * BLK, ROW_W), jnp.float32),
        pltpu.VMEM((RPT, ROW_W), jnp.float32),
        pltpu.VMEM_SHARED((NP, ROW_W), jnp.float32),
        pltpu.VMEM_SHARED((NP, ROW_W), jnp.float32),
        pltpu.SemaphoreType.DMA,
    ],
    compiler_params=_SC_PARAMS,
)
def _scatter_edges(ea_hbm, sidx_hbm, ridx_hbm, zeros_hbm, out_hbm,
                   sidx_v, ridx_v, vbuf, tmp, accs_sh, accr_sh, sem_st):
    cid = lax.axis_index("c")
    sid = lax.axis_index("s")
    w = _worker_id()
    z0 = sid * RPT
    pltpu.sync_copy(zeros_hbm.at[pl.ds(z0, RPT)], accs_sh.at[pl.ds(z0, RPT)])
    pltpu.sync_copy(zeros_hbm.at[pl.ds(z0, RPT)], accr_sh.at[pl.ds(z0, RPT)])
    pltpu.sync_copy(sidx_hbm.at[pl.ds(w * BPW_A, BPW_A)], sidx_v)
    pltpu.sync_copy(ridx_hbm.at[pl.ds(w * BPW_A, BPW_A)], ridx_v)
    plsc.subcore_barrier()

    def group(g, carry):
        row0 = (w * BPW_A + g * GRP) * BLK
        pltpu.sync_copy(ea_hbm.at[pl.ds(row0, GRP * BLK)], vbuf)
        descs = []
        for bq in range(GRP):
            src = vbuf.at[pl.ds(bq * BLK, BLK)]
            j = g * GRP + bq
            descs.append(pltpu.async_copy(
                src, accs_sh.at[sidx_v.at[j]], sem_st, add=True))
            descs.append(pltpu.async_copy(
                src, accr_sh.at[ridx_v.at[j]], sem_st, add=True))
        for d in descs:
            d.wait()
        return carry

    lax.fori_loop(0, NGRP_A, group, 0)
    plsc.subcore_barrier()
    pltpu.sync_copy(accs_sh.at[pl.ds(z0, RPT)], tmp)
    pltpu.sync_copy(tmp, out_hbm.at[cid, 0, pl.ds(z0, RPT)])
    pltpu.sync_copy(accr_sh.at[pl.ds(z0, RPT)], tmp)
    pltpu.sync_copy(tmp, out_hbm.at[cid, 1, pl.ds(z0, RPT)])


@functools.partial(
    pl.kernel,
    out_type=jax.ShapeDtypeStruct((NUM_CORES, NP, ROW_W), jnp.float32),
    mesh=_MESH,
    scratch_types=[
        pltpu.VMEM((BPW_B, BLK), jnp.int32),
        pltpu.VMEM((BPW_B, BLK), jnp.int32),
        pltpu.VMEM((GRP * BLK, ROW_W), jnp.float32),
        pltpu.VMEM((RPT, ROW_W), jnp.float32),
        pltpu.VMEM_SHARED((NP, ROW_W), jnp.float32),
        pltpu.SemaphoreType.DMA,
        pltpu.SemaphoreType.DMA,
    ],
    compiler_params=_SC_PARAMS,
)
def _gather_scatter_add(agg_hbm, idxo_hbm, idxi_hbm, zeros_hbm, out_hbm,
                        idxo_v, idxi_v, gbuf, tmp, sums_sh, sem_g, sem_s):
    cid = lax.axis_index("c")
    sid = lax.axis_index("s")
    w = _worker_id()
    z0 = sid * RPT
    pltpu.sync_copy(zeros_hbm.at[pl.ds(z0, RPT)], sums_sh.at[pl.ds(z0, RPT)])
    pltpu.sync_copy(idxo_hbm.at[pl.ds(w * BPW_B, BPW_B)], idxo_v)
    pltpu.sync_copy(idxi_hbm.at[pl.ds(w * BPW_B, BPW_B)], idxi_v)
    plsc.subcore_barrier()

    def group(g, carry):
        descs = [
            pltpu.async_copy(agg_hbm.at[idxo_v.at[g * GRP + bq]],
                             gbuf.at[pl.ds(bq * BLK, BLK)], sem_g)
            for bq in range(GRP)
        ]
        for d in descs:
            d.wait()
        descs = [
            pltpu.async_copy(gbuf.at[pl.ds(bq * BLK, BLK)],
                             sums_sh.at[idxi_v.at[g * GRP + bq]],
                             sem_s, add=True)
            for bq in range(GRP)
        ]
        for d in descs:
            d.wait()
        return carry

    lax.fori_loop(0, NGRP_B, group, 0)
    plsc.subcore_barrier()
    pltpu.sync_copy(sums_sh.at[pl.ds(z0, RPT)], tmp)
    pltpu.sync_copy(tmp, out_hbm.at[cid, pl.ds(z0, RPT)])


ROWS_BLK = 400


def _dense_body(x_ref, s0_ref, s1_ref, w_ref, b_ref, out_ref):
    sums = s0_ref[...] + s1_ref[...]
    cnt = jnp.maximum(sums[:, HALF:HALF + 1], 1.0)
    navg = sums[:, :HALF] / cnt
    xb = x_ref[...]
    w = w_ref[...]
    out = jnp.dot(navg, w[:HALF, :], preferred_element_type=jnp.float32)
    out += jnp.dot(xb, w[HALF:, :], preferred_element_type=jnp.float32)
    out_ref[...] = out + b_ref[...]


def kernel(x, edge_attr, W, b, edge_index):
    senders = edge_index[0]
    receivers = edge_index[1]
    sidx = senders.reshape(NBLK_A, BLK)
    ridx = receivers.reshape(NBLK_A, BLK)
    zeros_np = jnp.zeros((NP, ROW_W), dtype=jnp.float32)

    acc = _scatter_edges(edge_attr, sidx, ridx, zeros_np)
    agg8 = (acc[0, 0, :N_NODES, :HALF] + acc[1, 0, :N_NODES, :HALF]
            + acc[0, 1, :N_NODES, HALF:] + acc[1, 1, :N_NODES, HALF:])
    agg = jnp.concatenate(
        [agg8,
         jnp.ones((N_NODES, 1), dtype=jnp.float32),
         jnp.zeros((N_NODES, ROW_W - HALF - 1), dtype=jnp.float32)], axis=1)

    idx_in = jnp.concatenate([senders, receivers]).reshape(NBLK_B, BLK)
    idx_out = jnp.concatenate([receivers, senders]).reshape(NBLK_B, BLK)
    sums_pair = _gather_scatter_add(agg, idx_out, idx_in, zeros_np)

    grid = N_NODES // ROWS_BLK
    out = pl.pallas_call(
        _dense_body,
        grid=(grid,),
        in_specs=[
            pl.BlockSpec((ROWS_BLK, D_FEAT), lambda i: (i, 0)),
            pl.BlockSpec((ROWS_BLK, ROW_W), lambda i: (i, 0)),
            pl.BlockSpec((ROWS_BLK, ROW_W), lambda i: (i, 0)),
            pl.BlockSpec((D_FEAT + HALF, D_FEAT), lambda i: (0, 0)),
            pl.BlockSpec((1, D_FEAT), lambda i: (0, 0)),
        ],
        out_specs=pl.BlockSpec((ROWS_BLK, D_FEAT), lambda i: (i, 0)),
        out_shape=jax.ShapeDtypeStruct((N_NODES, D_FEAT), jnp.float32),
    )(x, sums_pair[0, :N_NODES], sums_pair[1, :N_NODES], W,
      b.reshape(1, D_FEAT))
    return out


# speedup vs baseline: 12.8326x; 1.1516x over previous
import functools

import jax
import jax.numpy as jnp
from jax import lax
from jax.experimental import pallas as pl
from jax.experimental.pallas import tpu as pltpu
from jax.experimental.pallas import tpu_sc as plsc

N_NODES = 10000
N_EDGES = 160000
TWO_E = 2 * N_EDGES
D_FEAT = 256
HALF = 8
ROW_W = 16
LANES = 16

NUM_CORES = 2
NUM_SUBCORES = 16
NUM_WORKERS = NUM_CORES * NUM_SUBCORES
BLK = 125
GRP = 8

BPW_A = N_EDGES // (NUM_WORKERS * BLK)
NBLK_A = N_EDGES // BLK
NGRP_A = BPW_A // GRP

BPW_B = TWO_E // (NUM_WORKERS * BLK)
NBLK_B = TWO_E // BLK
NGRP_B = BPW_B // GRP

NP = N_NODES
RPT = NP // NUM_SUBCORES


def _worker_id():
    return lax.axis_index("c") * NUM_SUBCORES + lax.axis_index("s")


_MESH = plsc.VectorSubcoreMesh(core_axis_name="c", subcore_axis_name="s")
_SC_PARAMS = pltpu.CompilerParams(use_tc_tiling_on_sc=False,
                                  needs_layout_passes=False)


@functools.partial(
    pl.kernel,
    out_type=jax.ShapeDtypeStruct((NUM_CORES, 2, NP, ROW_W), jnp.float32),
    mesh=_MESH,
    scratch_types=[
        pltpu.VMEM((BPW_A, BLK), jnp.int32),
        pltpu.VMEM((BPW_A, BLK), jnp.int32),
        pltpu.VMEM((GRP * BLK, ROW_W), jnp.float32),
        pltpu.VMEM((RPT, ROW_W), jnp.float32),
        pltpu.VMEM_SHARED((NP, ROW_W), jnp.float32),
        pltpu.VMEM_SHARED((NP, ROW_W), jnp.float32),
        pltpu.SemaphoreType.DMA,
    ],
    compiler_params=_SC_PARAMS,
)
def _scatter_edges(ea_hbm, sidx_hbm, ridx_hbm, zeros_hbm, out_hbm,
                   sidx_v, ridx_v, vbuf, tmp, accs_sh, accr_sh, sem_st):
    cid = lax.axis_index("c")
    sid = lax.axis_index("s")
    w = _worker_id()
    z0 = sid * RPT
    pltpu.sync_copy(zeros_hbm.at[pl.ds(z0, RPT)], accs_sh.at[pl.ds(z0, RPT)])
    pltpu.sync_copy(zeros_hbm.at[pl.ds(z0, RPT)], accr_sh.at[pl.ds(z0, RPT)])
    pltpu.sync_copy(sidx_hbm.at[pl.ds(w * BPW_A, BPW_A)], sidx_v)
    pltpu.sync_copy(ridx_hbm.at[pl.ds(w * BPW_A, BPW_A)], ridx_v)
    plsc.subcore_barrier()

    def group(g, carry):
        row0 = (w * BPW_A + g * GRP) * BLK
        pltpu.sync_copy(ea_hbm.at[pl.ds(row0, GRP * BLK)], vbuf)
        descs = []
        for bq in range(GRP):
            src = vbuf.at[pl.ds(bq * BLK, BLK)]
            j = g * GRP + bq
            descs.append(pltpu.async_copy(
                src, accs_sh.at[sidx_v.at[j]], sem_st, add=True))
            descs.append(pltpu.async_copy(
                src, accr_sh.at[ridx_v.at[j]], sem_st, add=True))
        for d in descs:
            d.wait()
        return carry

    lax.fori_loop(0, NGRP_A, group, 0)
    plsc.subcore_barrier()
    pltpu.sync_copy(accs_sh.at[pl.ds(z0, RPT)], tmp)
    pltpu.sync_copy(tmp, out_hbm.at[cid, 0, pl.ds(z0, RPT)])
    pltpu.sync_copy(accr_sh.at[pl.ds(z0, RPT)], tmp)
    pltpu.sync_copy(tmp, out_hbm.at[cid, 1, pl.ds(z0, RPT)])


@functools.partial(
    pl.kernel,
    out_type=jax.ShapeDtypeStruct((NUM_CORES, NP, ROW_W), jnp.float32),
    mesh=_MESH,
    scratch_types=[
        pltpu.VMEM((BPW_B, BLK), jnp.int32),
        pltpu.VMEM((BPW_B, BLK), jnp.int32),
        pltpu.VMEM((GRP * BLK, ROW_W), jnp.float32),
        pltpu.VMEM((RPT, ROW_W), jnp.float32),
        pltpu.VMEM((RPT, ROW_W), jnp.float32),
        pltpu.VMEM((RPT, ROW_W), jnp.float32),
        pltpu.VMEM((RPT, ROW_W), jnp.float32),
        pltpu.VMEM((RPT, ROW_W), jnp.float32),
        pltpu.VMEM_SHARED((NP, ROW_W), jnp.float32),
        pltpu.VMEM_SHARED((NP, ROW_W), jnp.float32),
        pltpu.SemaphoreType.DMA,
        pltpu.SemaphoreType.DMA,
    ],
    compiler_params=_SC_PARAMS,
)
def _gather_scatter_add(acc_hbm, idxo_hbm, idxi_hbm, zeros_hbm, out_hbm,
                        idxo_v, idxi_v, gbuf, bs0, bs1, br0, br1, aggbuf,
                        agg_sh, sums_sh, sem_g, sem_s):
    cid = lax.axis_index("c")
    sid = lax.axis_index("s")
    w = _worker_id()
    z0 = sid * RPT
    pltpu.sync_copy(zeros_hbm.at[pl.ds(z0, RPT)], sums_sh.at[pl.ds(z0, RPT)])
    pltpu.sync_copy(idxo_hbm.at[pl.ds(w * BPW_B, BPW_B)], idxo_v)
    pltpu.sync_copy(idxi_hbm.at[pl.ds(w * BPW_B, BPW_B)], idxi_v)
    pltpu.sync_copy(acc_hbm.at[0, 0, pl.ds(z0, RPT)], bs0)
    pltpu.sync_copy(acc_hbm.at[1, 0, pl.ds(z0, RPT)], bs1)
    pltpu.sync_copy(acc_hbm.at[0, 1, pl.ds(z0, RPT)], br0)
    pltpu.sync_copy(acc_hbm.at[1, 1, pl.ds(z0, RPT)], br1)
    lane = lax.iota(jnp.int32, LANES)
    perm = lane ^ HALF
    tail = jnp.where(lane == HALF, 1.0, 0.0)
    lo = lane < HALF

    def comb(i, carry):
        srow = bs0[i] + bs1[i]
        spl = jnp.full((LANES,), i, dtype=jnp.int32)
        rrot = (plsc.load_gather(br0, [spl, perm])
                + plsc.load_gather(br1, [spl, perm]))
        aggbuf[i] = jnp.where(lo, srow + rrot, tail)
        return carry

    lax.fori_loop(0, RPT, comb, 0)
    pltpu.sync_copy(aggbuf, agg_sh.at[pl.ds(z0, RPT)])
    plsc.subcore_barrier()

    def group(g, carry):
        descs = [
            pltpu.async_copy(agg_sh.at[idxo_v.at[g * GRP + bq]],
                             gbuf.at[pl.ds(bq * BLK, BLK)], sem_g)
            for bq in range(GRP)
        ]
        for d in descs:
            d.wait()
        descs = [
            pltpu.async_copy(gbuf.at[pl.ds(bq * BLK, BLK)],
                             sums_sh.at[idxi_v.at[g * GRP + bq]],
                             sem_s, add=True)
            for bq in range(GRP)
        ]
        for d in descs:
            d.wait()
        return carry

    lax.fori_loop(0, NGRP_B, group, 0)
    plsc.subcore_barrier()
    pltpu.sync_copy(sums_sh.at[pl.ds(z0, RPT)], tmp := bs0)
    pltpu.sync_copy(tmp, out_hbm.at[cid, pl.ds(z0, RPT)])


ROWS_BLK = 400


def _dense_body(x_ref, w_ref, b_ref, out_ref):
    out_ref[...] = jnp.dot(
        x_ref[...], w_ref[HALF:, :],
        preferred_element_type=jnp.float32) + b_ref[...]


def _finish_body(p1_ref, s0_ref, s1_ref, w_ref, out_ref):
    sums = s0_ref[0] + s1_ref[0]
    cnt = jnp.maximum(sums[:, HALF:HALF + 1], 1.0)
    navg = sums[:, :HALF] / cnt
    out_ref[...] = p1_ref[...] + jnp.dot(
        navg, w_ref[:HALF, :], preferred_element_type=jnp.float32)


def kernel(x, edge_attr, W, b, edge_index):
    senders = edge_index[0]
    receivers = edge_index[1]
    sidx = senders.reshape(NBLK_A, BLK)
    ridx = receivers.reshape(NBLK_A, BLK)
    zeros_np = jnp.zeros((NP, ROW_W), dtype=jnp.float32)

    grid = N_NODES // ROWS_BLK
    part1 = pl.pallas_call(
        _dense_body,
        grid=(grid,),
        in_specs=[
            pl.BlockSpec((ROWS_BLK, D_FEAT), lambda i: (i, 0)),
            pl.BlockSpec((D_FEAT + HALF, D_FEAT), lambda i: (0, 0)),
            pl.BlockSpec((1, D_FEAT), lambda i: (0, 0)),
        ],
        out_specs=pl.BlockSpec((ROWS_BLK, D_FEAT), lambda i: (i, 0)),
        out_shape=jax.ShapeDtypeStruct((N_NODES, D_FEAT), jnp.float32),
    )(x, W, b.reshape(1, D_FEAT))

    acc = _scatter_edges(edge_attr, sidx, ridx, zeros_np)

    idx_in = jnp.concatenate([senders, receivers]).reshape(NBLK_B, BLK)
    idx_out = jnp.concatenate([receivers, senders]).reshape(NBLK_B, BLK)
    sums_pair = _gather_scatter_add(acc, idx_out, idx_in, zeros_np)

    out = pl.pallas_call(
        _finish_body,
        grid=(grid,),
        in_specs=[
            pl.BlockSpec((ROWS_BLK, D_FEAT), lambda i: (i, 0)),
            pl.BlockSpec((1, ROWS_BLK, ROW_W), lambda i: (0, i, 0)),
            pl.BlockSpec((1, ROWS_BLK, ROW_W), lambda i: (1, i, 0)),
            pl.BlockSpec((D_FEAT + HALF, D_FEAT), lambda i: (0, 0)),
        ],
        out_specs=pl.BlockSpec((ROWS_BLK, D_FEAT), lambda i: (i, 0)),
        out_shape=jax.ShapeDtypeStruct((N_NODES, D_FEAT), jnp.float32),
    )(part1, sums_pair, sums_pair, W)
    return out
